# Initial kernel scaffold; baseline (speedup 1.0000x reference)
import jax
import jax.numpy as jnp
from jax.experimental import pallas as pl


def kernel(x, gate_w, w1, w2, w3):
    raise NotImplementedError("write your pallas kernel here")


# speedup vs baseline: 1.2382x; 1.2382x over previous
import functools

import jax
import jax.numpy as jnp
from jax import lax
from jax.experimental import pallas as pl
from jax.experimental.pallas import tpu as pltpu
from jax.experimental.pallas import tpu_sc as plsc

D = 1024
H = 2048
E = 8
TOPK = 2
T = 2048
P = T * TOPK
BM = 256
NPAD = P + E * BM
NB = NPAD // BM
BH = 512
NH = H // BH
CHUNK = 512

NC = 2
NS = 16
NW = NC * NS
PPW = P // NW
CH = 64


def _router_body(x_ref, gw_ref, wts_ref, dest_ref, meta_ref):
    xx = x_ref[...]
    gw = gw_ref[...]
    logits = lax.dot_general(xx, gw, (((1,), (1,)), ((), ())),
                             preferred_element_type=jnp.float32)
    col = lax.broadcasted_iota(jnp.int32, (T, E), 1)
    m1 = jnp.max(logits, axis=1, keepdims=True)
    i1 = jnp.min(jnp.where(logits == m1, col, E), axis=1, keepdims=True)
    masked = jnp.where(col == i1, -jnp.inf, logits)
    m2 = jnp.max(masked, axis=1, keepdims=True)
    i2 = jnp.min(jnp.where(masked == m2, col, E), axis=1, keepdims=True)
    w_first = 1.0 / (1.0 + jnp.exp(m2 - m1))
    wts_ref[...] = jnp.concatenate([w_first, 1.0 - w_first], axis=1)

    e_pairs = jnp.concatenate([i1, i2], axis=0)
    colp = lax.broadcasted_iota(jnp.int32, (P, E), 1)
    onehot = (e_pairs == colp).astype(jnp.float32)

    r = lax.broadcasted_iota(jnp.int32, (CHUNK, CHUNK), 0)
    c = lax.broadcasted_iota(jnp.int32, (CHUNK, CHUNK), 1)
    tri = (r > c).astype(jnp.float32)
    totals = jnp.zeros((1, E), jnp.float32)
    granks = []
    for ci in range(P // CHUNK):
        oc = onehot[ci * CHUNK:(ci + 1) * CHUNK]
        ranks = lax.dot_general(tri, oc, (((1,), (0,)), ((), ())),
                                preferred_element_type=jnp.float32) + totals
        granks.append(jnp.sum(oc * ranks, axis=1, keepdims=True))
        totals = totals + jnp.sum(oc, axis=0, keepdims=True)
    grank = jnp.concatenate(granks, axis=0)

    counts = totals.astype(jnp.int32)
    cpad = ((counts + BM - 1) // BM) * BM
    offs_list = [jnp.zeros((1, 1), jnp.int32)]
    for e in range(1, E):
        offs_list.append(offs_list[-1] + cpad[:, e - 1:e])
    offs = jnp.concatenate(offs_list, axis=1)

    dest = grank + jnp.sum(onehot * offs.astype(jnp.float32),
                           axis=1, keepdims=True)
    dest_ref[...] = dest.astype(jnp.int32)

    nact = jnp.sum(cpad) // BM
    brow = lax.broadcasted_iota(jnp.int32, (32, E), 0)
    bm_rows = jnp.minimum(brow, nact - 1) * BM
    be = jnp.sum((bm_rows >= offs).astype(jnp.int32), axis=1, keepdims=True) - 1
    rowid = lax.broadcasted_iota(jnp.int32, (32, 1), 0)
    meta_ref[...] = jnp.where(rowid == NB, nact, be)


def _router_call(flat, gate_w):
    return pl.pallas_call(
        _router_body,
        out_shape=(
            jax.ShapeDtypeStruct((T, TOPK), jnp.float32),
            jax.ShapeDtypeStruct((P, 1), jnp.int32),
            jax.ShapeDtypeStruct((32, 1), jnp.int32),
        ),
    )(flat, gate_w)


def _sc_mesh():
    return plsc.VectorSubcoreMesh(core_axis_name="c", subcore_axis_name="s")


def _dispatch_body(x_hbm, tok_hbm, dest_hbm, xs_hbm, idx_v, dst_v, rows_v, sem):
    wid = lax.axis_index("s") * NC + lax.axis_index("c")
    for half in range(PPW // CH):
        base = wid * PPW + half * CH
        pltpu.sync_copy(tok_hbm.at[pl.ds(base, CH)], idx_v)
        pltpu.sync_copy(dest_hbm.at[pl.ds(base, CH)], dst_v)
        pltpu.async_copy(x_hbm.at[idx_v], rows_v, sem).wait()
        pltpu.async_copy(rows_v, xs_hbm.at[dst_v], sem).wait()


def _dispatch_call(flat, tok, dest):
    f = functools.partial(
        pl.kernel,
        mesh=_sc_mesh(),
        out_type=jax.ShapeDtypeStruct((NPAD, D), jnp.float32),
        scratch_types=[
            pltpu.VMEM((CH,), jnp.int32),
            pltpu.VMEM((CH,), jnp.int32),
            pltpu.VMEM((CH, D), jnp.float32),
            pltpu.SemaphoreType.DMA,
        ],
    )(_dispatch_body)
    return f(flat, tok, dest)


def _collect_body(eo_hbm, dest_hbm, geo_hbm, idx_v, rows_v, sem):
    wid = lax.axis_index("s") * NC + lax.axis_index("c")
    for half in range(PPW // CH):
        base = wid * PPW + half * CH
        pltpu.sync_copy(dest_hbm.at[pl.ds(base, CH)], idx_v)
        pltpu.async_copy(eo_hbm.at[idx_v], rows_v, sem).wait()
        pltpu.sync_copy(rows_v, geo_hbm.at[pl.ds(base, CH)])


def _collect_call(eo, dest):
    f = functools.partial(
        pl.kernel,
        mesh=_sc_mesh(),
        out_type=jax.ShapeDtypeStruct((P, D), jnp.float32),
        scratch_types=[
            pltpu.VMEM((CH,), jnp.int32),
            pltpu.VMEM((CH, D), jnp.float32),
            pltpu.SemaphoreType.DMA,
        ],
    )(_collect_body)
    return f(eo, dest)


def _ffn_body(meta_ref, xs_ref, w1_ref, w3_ref, w2_ref, out_ref, acc_ref):
    b = pl.program_id(0)
    h = pl.program_id(1)
    nact = meta_ref[NB]

    @pl.when(b < nact)
    def _():
        xb = xs_ref[...]
        w1t = w1_ref[0]
        w3t = w3_ref[0]
        w2t = w2_ref[0]
        h1 = lax.dot_general(xb, w1t, (((1,), (1,)), ((), ())),
                             preferred_element_type=jnp.float32)
        h3 = lax.dot_general(xb, w3t, (((1,), (1,)), ((), ())),
                             preferred_element_type=jnp.float32)
        hh = h1 * (1.0 / (1.0 + jnp.exp(-h1))) * h3
        part = lax.dot_general(hh, w2t, (((1,), (1,)), ((), ())),
                               preferred_element_type=jnp.float32)

        @pl.when(h == 0)
        def _():
            acc_ref[...] = part

        @pl.when(h > 0)
        def _():
            acc_ref[...] += part

        @pl.when(h == NH - 1)
        def _():
            out_ref[...] = acc_ref[...]


def _ffn_call(meta, xs, w1, w3, w2):
    grid_spec = pltpu.PrefetchScalarGridSpec(
        num_scalar_prefetch=1,
        grid=(NB, NH),
        in_specs=[
            pl.BlockSpec((BM, D), lambda b, h, m: (b, 0)),
            pl.BlockSpec((1, BH, D), lambda b, h, m: (m[b], h, 0)),
            pl.BlockSpec((1, BH, D), lambda b, h, m: (m[b], h, 0)),
            pl.BlockSpec((1, D, BH), lambda b, h, m: (m[b], 0, h)),
        ],
        out_specs=pl.BlockSpec((BM, D), lambda b, h, m: (b, 0)),
        scratch_shapes=[pltpu.VMEM((BM, D), jnp.float32)],
    )
    return pl.pallas_call(
        _ffn_body,
        grid_spec=grid_spec,
        out_shape=jax.ShapeDtypeStruct((NPAD, D), jnp.float32),
        compiler_params=pltpu.CompilerParams(
            dimension_semantics=("arbitrary", "arbitrary"),
        ),
    )(meta, xs, w1, w3, w2)


BT = 512


def _combine_body(g0_ref, g1_ref, w_ref, out_ref):
    w = w_ref[...]
    out_ref[...] = g0_ref[...] * w[:, 0:1] + g1_ref[...] * w[:, 1:2]


def _combine_call(geo, wts):
    return pl.pallas_call(
        _combine_body,
        grid=(T // BT,),
        in_specs=[
            pl.BlockSpec((BT, D), lambda i: (i, 0)),
            pl.BlockSpec((BT, D), lambda i: (i + T // BT, 0)),
            pl.BlockSpec((BT, TOPK), lambda i: (i, 0)),
        ],
        out_specs=pl.BlockSpec((BT, D), lambda i: (i, 0)),
        out_shape=jax.ShapeDtypeStruct((T, D), jnp.float32),
    )(geo, geo, wts)


def kernel(x, gate_w, w1, w2, w3):
    B, T_, D_ = x.shape
    flat = x.reshape(T, D)
    wts, dest2, meta2 = _router_call(flat, gate_w)
    dest = dest2.reshape(P)
    meta = meta2.reshape(32)
    tok = (jnp.arange(P, dtype=jnp.int32) % T)
    xs = _dispatch_call(flat, tok, dest)
    eo = _ffn_call(meta, xs, w1, w3, w2)
    geo = _collect_call(eo, dest)
    out = _combine_call(geo, wts)
    return out.reshape(B, T_, D_)


# speedup vs baseline: 1.5885x; 1.2829x over previous
import functools

import jax
import jax.numpy as jnp
from jax import lax
from jax.experimental import pallas as pl
from jax.experimental.pallas import tpu as pltpu
from jax.experimental.pallas import tpu_sc as plsc

D = 1024
H = 2048
E = 8
TOPK = 2
T = 2048
P = T * TOPK
BM = 256
NPAD = P + E * BM
NB = NPAD // BM
BH = 512
NH = H // BH
CHUNK = 512

NC = 2
NS = 16
NW = NC * NS
PPW = P // NW
CH = 64


def _router_body(x_ref, gw_ref, wts_ref, dest_ref, meta_ref):
    xx = x_ref[...]
    gw = gw_ref[...]
    logits = lax.dot_general(xx, gw, (((1,), (1,)), ((), ())),
                             preferred_element_type=jnp.float32)
    col = lax.broadcasted_iota(jnp.int32, (T, E), 1)
    m1 = jnp.max(logits, axis=1, keepdims=True)
    i1 = jnp.min(jnp.where(logits == m1, col, E), axis=1, keepdims=True)
    masked = jnp.where(col == i1, -jnp.inf, logits)
    m2 = jnp.max(masked, axis=1, keepdims=True)
    i2 = jnp.min(jnp.where(masked == m2, col, E), axis=1, keepdims=True)
    w_first = 1.0 / (1.0 + jnp.exp(m2 - m1))
    wts_ref[...] = jnp.concatenate([w_first, 1.0 - w_first], axis=1)

    e_pairs = jnp.concatenate([i1, i2], axis=0)
    colp = lax.broadcasted_iota(jnp.int32, (P, E), 1)
    onehot = (e_pairs == colp).astype(jnp.float32)

    r = lax.broadcasted_iota(jnp.int32, (CHUNK, CHUNK), 0)
    c = lax.broadcasted_iota(jnp.int32, (CHUNK, CHUNK), 1)
    tri = (r > c).astype(jnp.float32)
    totals = jnp.zeros((1, E), jnp.float32)
    granks = []
    for ci in range(P // CHUNK):
        oc = onehot[ci * CHUNK:(ci + 1) * CHUNK]
        ranks = lax.dot_general(tri, oc, (((1,), (0,)), ((), ())),
                                preferred_element_type=jnp.float32) + totals
        granks.append(jnp.sum(oc * ranks, axis=1, keepdims=True))
        totals = totals + jnp.sum(oc, axis=0, keepdims=True)
    grank = jnp.concatenate(granks, axis=0)

    counts = totals.astype(jnp.int32)
    cpad = ((counts + BM - 1) // BM) * BM
    offs_list = [jnp.zeros((1, 1), jnp.int32)]
    for e in range(1, E):
        offs_list.append(offs_list[-1] + cpad[:, e - 1:e])
    offs = jnp.concatenate(offs_list, axis=1)

    dest = grank + jnp.sum(onehot * offs.astype(jnp.float32),
                           axis=1, keepdims=True)
    dest_ref[...] = dest.astype(jnp.int32)

    nact = jnp.sum(cpad) // BM
    brow = lax.broadcasted_iota(jnp.int32, (32, E), 0)
    bm_rows = jnp.minimum(brow, nact - 1) * BM
    be = jnp.sum((bm_rows >= offs).astype(jnp.int32), axis=1, keepdims=True) - 1
    rowid = lax.broadcasted_iota(jnp.int32, (32, 1), 0)
    meta_ref[...] = jnp.where(rowid == NB, nact, be)


def _router_call(flat, gate_w):
    return pl.pallas_call(
        _router_body,
        out_shape=(
            jax.ShapeDtypeStruct((T, TOPK), jnp.float32),
            jax.ShapeDtypeStruct((P, 1), jnp.int32),
            jax.ShapeDtypeStruct((32, 1), jnp.int32),
        ),
    )(flat, gate_w)


def _sc_mesh():
    return plsc.VectorSubcoreMesh(core_axis_name="c", subcore_axis_name="s")


def _dispatch_body(x_hbm, tok_hbm, dest_hbm, xs_hbm, idx_v, dst_v, rows_v, sem):
    wid = lax.axis_index("s") * NC + lax.axis_index("c")
    for half in range(PPW // CH):
        base = wid * PPW + half * CH
        pltpu.sync_copy(tok_hbm.at[pl.ds(base, CH)], idx_v)
        pltpu.sync_copy(dest_hbm.at[pl.ds(base, CH)], dst_v)
        pltpu.async_copy(x_hbm.at[idx_v], rows_v, sem).wait()
        pltpu.async_copy(rows_v, xs_hbm.at[dst_v], sem).wait()


def _dispatch_call(flat, tok, dest):
    f = functools.partial(
        pl.kernel,
        mesh=_sc_mesh(),
        out_type=jax.ShapeDtypeStruct((NPAD, D), jnp.float32),
        scratch_types=[
            pltpu.VMEM((CH,), jnp.int32),
            pltpu.VMEM((CH,), jnp.int32),
            pltpu.VMEM((CH, D), jnp.float32),
            pltpu.SemaphoreType.DMA,
        ],
    )(_dispatch_body)
    return f(flat, tok, dest)


def _collect_body(eo_hbm, dest_hbm, geo_hbm, idx_v, rows_v, sem):
    wid = lax.axis_index("s") * NC + lax.axis_index("c")
    for half in range(PPW // CH):
        base = wid * PPW + half * CH
        pltpu.sync_copy(dest_hbm.at[pl.ds(base, CH)], idx_v)
        pltpu.async_copy(eo_hbm.at[idx_v], rows_v, sem).wait()
        pltpu.sync_copy(rows_v, geo_hbm.at[pl.ds(base, CH)])


def _collect_call(eo, dest):
    f = functools.partial(
        pl.kernel,
        mesh=_sc_mesh(),
        out_type=jax.ShapeDtypeStruct((P, D), jnp.float32),
        scratch_types=[
            pltpu.VMEM((CH,), jnp.int32),
            pltpu.VMEM((CH, D), jnp.float32),
            pltpu.SemaphoreType.DMA,
        ],
    )(_collect_body)
    return f(eo, dest)


def _ffn_up_body(meta_ref, xs_ref, w1_ref, w3_ref, hh_ref):
    b = pl.program_id(0)
    nact = meta_ref[NB]

    @pl.when(b < nact)
    def _():
        xb = xs_ref[...].astype(jnp.bfloat16)
        w1t = w1_ref[0].astype(jnp.bfloat16)
        w3t = w3_ref[0].astype(jnp.bfloat16)
        h1 = lax.dot_general(xb, w1t, (((1,), (1,)), ((), ())),
                             preferred_element_type=jnp.float32)
        h3 = lax.dot_general(xb, w3t, (((1,), (1,)), ((), ())),
                             preferred_element_type=jnp.float32)
        hh = h1 * (1.0 / (1.0 + jnp.exp(-h1))) * h3
        hh_ref[...] = hh.astype(jnp.bfloat16)


def _ffn_down_body(meta_ref, hh_ref, w2_ref, out_ref):
    b = pl.program_id(0)
    nact = meta_ref[NB]

    @pl.when(b < nact)
    def _():
        hh = hh_ref[...]
        w2t = w2_ref[0].astype(jnp.bfloat16)
        out_ref[...] = lax.dot_general(hh, w2t, (((1,), (1,)), ((), ())),
                                       preferred_element_type=jnp.float32)


def _ffn_call(meta, xs, w1, w3, w2):
    up_spec = pltpu.PrefetchScalarGridSpec(
        num_scalar_prefetch=1,
        grid=(NB,),
        in_specs=[
            pl.BlockSpec((BM, D), lambda b, m: (b, 0)),
            pl.BlockSpec((1, H, D), lambda b, m: (m[b], 0, 0)),
            pl.BlockSpec((1, H, D), lambda b, m: (m[b], 0, 0)),
        ],
        out_specs=pl.BlockSpec((BM, H), lambda b, m: (b, 0)),
    )
    hh = pl.pallas_call(
        _ffn_up_body,
        grid_spec=up_spec,
        out_shape=jax.ShapeDtypeStruct((NPAD, H), jnp.bfloat16),
        compiler_params=pltpu.CompilerParams(
            dimension_semantics=("arbitrary",),
        ),
    )(meta, xs, w1, w3)

    down_spec = pltpu.PrefetchScalarGridSpec(
        num_scalar_prefetch=1,
        grid=(NB,),
        in_specs=[
            pl.BlockSpec((BM, H), lambda b, m: (b, 0)),
            pl.BlockSpec((1, D, H), lambda b, m: (m[b], 0, 0)),
        ],
        out_specs=pl.BlockSpec((BM, D), lambda b, m: (b, 0)),
    )
    return pl.pallas_call(
        _ffn_down_body,
        grid_spec=down_spec,
        out_shape=jax.ShapeDtypeStruct((NPAD, D), jnp.float32),
        compiler_params=pltpu.CompilerParams(
            dimension_semantics=("arbitrary",),
        ),
    )(meta, hh, w2)


BT = 512


def _combine_body(g0_ref, g1_ref, w_ref, out_ref):
    w = w_ref[...]
    out_ref[...] = g0_ref[...] * w[:, 0:1] + g1_ref[...] * w[:, 1:2]


def _combine_call(geo, wts):
    return pl.pallas_call(
        _combine_body,
        grid=(T // BT,),
        in_specs=[
            pl.BlockSpec((BT, D), lambda i: (i, 0)),
            pl.BlockSpec((BT, D), lambda i: (i + T // BT, 0)),
            pl.BlockSpec((BT, TOPK), lambda i: (i, 0)),
        ],
        out_specs=pl.BlockSpec((BT, D), lambda i: (i, 0)),
        out_shape=jax.ShapeDtypeStruct((T, D), jnp.float32),
    )(geo, geo, wts)


def kernel(x, gate_w, w1, w2, w3):
    B, T_, D_ = x.shape
    flat = x.reshape(T, D)
    wts, dest2, meta2 = _router_call(flat, gate_w)
    dest = dest2.reshape(P)
    meta = meta2.reshape(32)
    tok = (jnp.arange(P, dtype=jnp.int32) % T)
    xs = _dispatch_call(flat, tok, dest)
    eo = _ffn_call(meta, xs, w1, w3, w2)
    geo = _collect_call(eo, dest)
    out = _combine_call(geo, wts)
    return out.reshape(B, T_, D_)
